# Initial kernel scaffold; baseline (speedup 1.0000x reference)
import jax
import jax.numpy as jnp
from jax.experimental import pallas as pl


def kernel(x, weight):
    raise NotImplementedError("write your pallas kernel here")


# speedup vs baseline: 18.4373x; 18.4373x over previous
import functools

import jax
import jax.numpy as jnp
from jax.experimental import pallas as pl

_BLK = 256
_K = 512
_D = 256


def _vq_block(x_ref, emb_ref, embT_ref, out_ref):
    x = x_ref[...]
    emb = emb_ref[...]
    embT = embT_ref[...]

    esq = jnp.sum(emb * emb, axis=0)
    dots = jax.lax.dot_general(
        x, emb, (((1,), (0,)), ((), ())),
        precision=jax.lax.Precision.HIGHEST,
        preferred_element_type=jnp.float32)
    s = esq[None, :] - 2.0 * dots

    kidx = jax.lax.broadcasted_iota(jnp.int32, s.shape, 1)
    m1 = jnp.min(s, axis=1, keepdims=True)
    i1 = jnp.min(jnp.where(s == m1, kidx, _K), axis=1)
    s2 = jnp.where(kidx == i1[:, None], jnp.inf, s)
    m2 = jnp.min(s2, axis=1, keepdims=True)
    i2 = jnp.min(jnp.where(s2 == m2, kidx, _K), axis=1)

    oh1 = (kidx == i1[:, None]).astype(jnp.float32)
    oh2 = (kidx == i2[:, None]).astype(jnp.float32)
    e1 = jax.lax.dot_general(
        oh1, embT, (((1,), (0,)), ((), ())),
        precision=jax.lax.Precision.HIGHEST,
        preferred_element_type=jnp.float32)
    e2 = jax.lax.dot_general(
        oh2, embT, (((1,), (0,)), ((), ())),
        precision=jax.lax.Precision.HIGHEST,
        preferred_element_type=jnp.float32)

    d1 = jnp.sum((x - e1) ** 2, axis=1)
    d2 = jnp.sum((x - e2) ** 2, axis=1)
    pick1 = (d1 < d2) | ((d1 == d2) & (i1 < i2))
    out_ref[...] = jnp.where(pick1[:, None], e1, e2)


@jax.jit
def kernel(x, weight):
    b = x.shape[0]
    grid = (b // _BLK,)
    return pl.pallas_call(
        _vq_block,
        grid=grid,
        in_specs=[
            pl.BlockSpec((_BLK, _D), lambda i: (i, 0)),
            pl.BlockSpec((_D, _K), lambda i: (0, 0)),
            pl.BlockSpec((_K, _D), lambda i: (0, 0)),
        ],
        out_specs=pl.BlockSpec((_BLK, _D), lambda i: (i, 0)),
        out_shape=jax.ShapeDtypeStruct((b, _D), jnp.float32),
    )(x, weight, weight.T)
